# baseline (speedup 1.0000x reference)
import jax
import jax.numpy as jnp
from jax.experimental import pallas as pl
from jax.experimental.pallas import tpu as pltpu

B, C, N, M, K = 4, 64, 8192, 2048, 3
NB = 256


def _conv_bn_kernel(ps_ref, w_ref, g_ref, b_ref, psc_ref):
    w = w_ref[...]
    ts = []
    s = jnp.zeros((C, 1), jnp.float32)
    ss = jnp.zeros((C, 1), jnp.float32)
    for b in range(B):
        t = jnp.dot(w, ps_ref[b], preferred_element_type=jnp.float32)
        ts.append(t)
        s = s + jnp.sum(t, axis=1, keepdims=True)
        ss = ss + jnp.sum(t * t, axis=1, keepdims=True)
    cnt = float(B * M)
    mean = s / cnt
    var = ss / cnt - mean * mean
    scale = g_ref[...] * jax.lax.rsqrt(var + 1e-5)
    for b in range(B):
        t = (ts[b] - mean) * scale + b_ref[...]
        psc_ref[b] = jnp.where(t >= 0, t, 0.2 * t)


def _knn_kernel(pcd_ref, ps_ref, psc_ref, rw_ref, y_ref, st_ref):
    q = pcd_ref[0]
    ps = ps_ref[0]
    qk = jax.lax.dot_general(q, ps, (((0,), (0,)), ((), ())),
                             preferred_element_type=jnp.float32)
    q2 = jnp.sum(q * q, axis=0)[:, None]
    k2 = jnp.sum(ps * ps, axis=0)[None, :]
    dist = jnp.maximum(q2 + k2 - 2.0 * qk, 0.0)

    inf = jnp.float32(jnp.inf)
    a = jnp.full((NB, 128), inf, jnp.float32)
    b = jnp.full((NB, 128), inf, jnp.float32)
    c = jnp.full((NB, 128), inf, jnp.float32)
    for i in range(M // 128):
        x = dist[:, i * 128:(i + 1) * 128]
        t = jnp.maximum(a, x)
        a = jnp.minimum(a, x)
        u = jnp.maximum(b, t)
        b = jnp.minimum(b, t)
        c = jnp.minimum(c, u)
    v1 = jnp.min(a, axis=1, keepdims=True)
    e1 = a == v1
    a2 = jnp.where(e1, b, a)
    b2 = jnp.where(e1, c, b)
    v2 = jnp.min(a2, axis=1, keepdims=True)
    a3 = jnp.where(a2 == v2, b2, a2)
    v3 = jnp.min(a3, axis=1, keepdims=True)

    wsum = (1.0 / (v1 + 1e-8) + 1.0 / (v2 + 1e-8) + 1.0 / (v3 + 1e-8))
    S = jnp.where(dist <= v3, 1.0 / (dist + 1e-8), 0.0)

    interp_t = jax.lax.dot_general(S, psc_ref[0], (((1,), (1,)), ((), ())),
                                   preferred_element_type=jnp.float32)
    interp = (interp_t / wsum).T

    rw = rw_ref[...]
    y = (jnp.dot(rw[:, :C], q, preferred_element_type=jnp.float32)
         + jnp.dot(rw[:, C:], interp, preferred_element_type=jnp.float32))
    y_ref[0] = y

    st = jnp.stack([jnp.sum(y, axis=1), jnp.sum(y * y, axis=1)], axis=1)
    first = pl.program_id(1) == 0

    @pl.when(first)
    def _():
        st_ref[0] = st

    @pl.when(jnp.logical_not(first))
    def _():
        st_ref[0] = st_ref[0] + st


def _bn2_kernel(y_ref, st_ref, g_ref, b_ref, o_ref):
    cnt = float(B * N)
    s = jnp.sum(st_ref[:, :, 0:1], axis=0)
    ss = jnp.sum(st_ref[:, :, 1:2], axis=0)
    mean = s / cnt
    var = ss / cnt - mean * mean
    scale = g_ref[...] * jax.lax.rsqrt(var + 1e-5)
    t = (y_ref[0] - mean) * scale + b_ref[...]
    o_ref[0] = jnp.where(t >= 0, t, 0.2 * t)


def kernel(pcd_up, points_select, idx_select, points_select_xyz, points_drop,
           idx_drop, pcd_up_xyz, conv_w, bn1_g, bn1_b, res_w, bn2_g, bn2_b):
    g1 = bn1_g.reshape(C, 1)
    b1 = bn1_b.reshape(C, 1)
    g2 = bn2_g.reshape(C, 1)
    b2 = bn2_b.reshape(C, 1)

    psc = pl.pallas_call(
        _conv_bn_kernel,
        out_shape=jax.ShapeDtypeStruct((B, C, M), jnp.float32),
    )(points_select, conv_w, g1, b1)

    J = N // NB
    y, st = pl.pallas_call(
        _knn_kernel,
        grid=(B, J),
        compiler_params=pltpu.CompilerParams(
            dimension_semantics=("parallel", "arbitrary")),
        in_specs=[
            pl.BlockSpec((1, C, NB), lambda b, j: (b, 0, j)),
            pl.BlockSpec((1, C, M), lambda b, j: (b, 0, 0)),
            pl.BlockSpec((1, C, M), lambda b, j: (b, 0, 0)),
            pl.BlockSpec((C, 2 * C), lambda b, j: (0, 0)),
        ],
        out_specs=[
            pl.BlockSpec((1, C, NB), lambda b, j: (b, 0, j)),
            pl.BlockSpec((1, C, 2), lambda b, j: (b, 0, 0)),
        ],
        out_shape=[
            jax.ShapeDtypeStruct((B, C, N), jnp.float32),
            jax.ShapeDtypeStruct((B, C, 2), jnp.float32),
        ],
    )(pcd_up, points_select, psc, res_w)

    x = pl.pallas_call(
        _bn2_kernel,
        grid=(B,),
        in_specs=[
            pl.BlockSpec((1, C, N), lambda b: (b, 0, 0)),
            pl.BlockSpec((B, C, 2), lambda b: (0, 0, 0)),
            pl.BlockSpec((C, 1), lambda b: (0, 0)),
            pl.BlockSpec((C, 1), lambda b: (0, 0)),
        ],
        out_specs=pl.BlockSpec((1, C, N), lambda b: (b, 0, 0)),
        out_shape=jax.ShapeDtypeStruct((B, C, N), jnp.float32),
    )(y, st, g2, b2)
    return x


# speedup vs baseline: 1.0537x; 1.0537x over previous
import jax
import jax.numpy as jnp
from jax.experimental import pallas as pl
from jax.experimental.pallas import tpu as pltpu

B, C, N, M, K = 4, 64, 8192, 2048, 3
NB = 256


def _conv_bn_kernel(ps_ref, w_ref, g_ref, b_ref, rw_ref, psc2_ref):
    w = w_ref[...]
    rw2 = rw_ref[:, C:]
    ts = []
    s = jnp.zeros((C, 1), jnp.float32)
    ss = jnp.zeros((C, 1), jnp.float32)
    for b in range(B):
        t = jnp.dot(w, ps_ref[b], preferred_element_type=jnp.float32)
        ts.append(t)
        s = s + jnp.sum(t, axis=1, keepdims=True)
        ss = ss + jnp.sum(t * t, axis=1, keepdims=True)
    cnt = float(B * M)
    mean = s / cnt
    var = ss / cnt - mean * mean
    scale = g_ref[...] * jax.lax.rsqrt(var + 1e-5)
    for b in range(B):
        t = (ts[b] - mean) * scale + b_ref[...]
        psc = jnp.where(t >= 0, t, 0.2 * t)
        psc2_ref[b] = jnp.dot(rw2, psc, preferred_element_type=jnp.float32)


def _knn_kernel(pcd_ref, ps_ref, psc2_ref, rw_ref, y_ref, st_ref):
    q = pcd_ref[0]
    ps = ps_ref[0]
    qk = jax.lax.dot_general(q, ps, (((0,), (0,)), ((), ())),
                             preferred_element_type=jnp.float32)
    q2 = jnp.sum(q * q, axis=0)[:, None]
    k2 = jnp.sum(ps * ps, axis=0)[None, :]
    dist = jnp.maximum(q2 + k2 - 2.0 * qk, 0.0)

    inf = jnp.float32(jnp.inf)
    a = jnp.full((NB, 128), inf, jnp.float32)
    b = jnp.full((NB, 128), inf, jnp.float32)
    c = jnp.full((NB, 128), inf, jnp.float32)
    for i in range(M // 128):
        x = dist[:, i * 128:(i + 1) * 128]
        t = jnp.maximum(a, x)
        a = jnp.minimum(a, x)
        u = jnp.maximum(b, t)
        b = jnp.minimum(b, t)
        c = jnp.minimum(c, u)
    v1 = jnp.min(a, axis=1, keepdims=True)
    e1 = a == v1
    a2 = jnp.where(e1, b, a)
    b2 = jnp.where(e1, c, b)
    v2 = jnp.min(a2, axis=1, keepdims=True)
    a3 = jnp.where(a2 == v2, b2, a2)
    v3 = jnp.min(a3, axis=1, keepdims=True)

    wsum = (1.0 / (v1 + 1e-8) + 1.0 / (v2 + 1e-8) + 1.0 / (v3 + 1e-8))
    S = jnp.where(dist <= v3, 1.0 / (dist + 1e-8), 0.0)

    y2t = jax.lax.dot_general(S, psc2_ref[0], (((1,), (1,)), ((), ())),
                              preferred_element_type=jnp.float32)
    rw = rw_ref[...]
    y = (jnp.dot(rw[:, :C], q, preferred_element_type=jnp.float32)
         + (y2t / wsum).T)
    y_ref[0] = y

    st = jnp.stack([jnp.sum(y, axis=1), jnp.sum(y * y, axis=1)], axis=1)
    first = pl.program_id(1) == 0

    @pl.when(first)
    def _():
        st_ref[0] = st

    @pl.when(jnp.logical_not(first))
    def _():
        st_ref[0] = st_ref[0] + st


def _bn2_kernel(y_ref, st_ref, g_ref, b_ref, o_ref):
    cnt = float(B * N)
    s = jnp.sum(st_ref[:, :, 0:1], axis=0)
    ss = jnp.sum(st_ref[:, :, 1:2], axis=0)
    mean = s / cnt
    var = ss / cnt - mean * mean
    scale = g_ref[...] * jax.lax.rsqrt(var + 1e-5)
    t = (y_ref[0] - mean) * scale + b_ref[...]
    o_ref[0] = jnp.where(t >= 0, t, 0.2 * t)


def kernel(pcd_up, points_select, idx_select, points_select_xyz, points_drop,
           idx_drop, pcd_up_xyz, conv_w, bn1_g, bn1_b, res_w, bn2_g, bn2_b):
    g1 = bn1_g.reshape(C, 1)
    b1 = bn1_b.reshape(C, 1)
    g2 = bn2_g.reshape(C, 1)
    b2 = bn2_b.reshape(C, 1)

    psc2 = pl.pallas_call(
        _conv_bn_kernel,
        out_shape=jax.ShapeDtypeStruct((B, C, M), jnp.float32),
    )(points_select, conv_w, g1, b1, res_w)

    J = N // NB
    y, st = pl.pallas_call(
        _knn_kernel,
        grid=(B, J),
        compiler_params=pltpu.CompilerParams(
            dimension_semantics=("parallel", "arbitrary")),
        in_specs=[
            pl.BlockSpec((1, C, NB), lambda b, j: (b, 0, j)),
            pl.BlockSpec((1, C, M), lambda b, j: (b, 0, 0)),
            pl.BlockSpec((1, C, M), lambda b, j: (b, 0, 0)),
            pl.BlockSpec((C, 2 * C), lambda b, j: (0, 0)),
        ],
        out_specs=[
            pl.BlockSpec((1, C, NB), lambda b, j: (b, 0, j)),
            pl.BlockSpec((1, C, 2), lambda b, j: (b, 0, 0)),
        ],
        out_shape=[
            jax.ShapeDtypeStruct((B, C, N), jnp.float32),
            jax.ShapeDtypeStruct((B, C, 2), jnp.float32),
        ],
    )(pcd_up, points_select, psc2, res_w)

    x = pl.pallas_call(
        _bn2_kernel,
        grid=(B,),
        in_specs=[
            pl.BlockSpec((1, C, N), lambda b: (b, 0, 0)),
            pl.BlockSpec((B, C, 2), lambda b: (0, 0, 0)),
            pl.BlockSpec((C, 1), lambda b: (0, 0)),
            pl.BlockSpec((C, 1), lambda b: (0, 0)),
        ],
        out_specs=pl.BlockSpec((1, C, N), lambda b: (b, 0, 0)),
        out_shape=jax.ShapeDtypeStruct((B, C, N), jnp.float32),
    )(y, st, g2, b2)
    return x


# speedup vs baseline: 1.2348x; 1.1719x over previous
import jax
import jax.numpy as jnp
from jax.experimental import pallas as pl
from jax.experimental.pallas import tpu as pltpu

B, C, N, M, K = 4, 64, 8192, 2048, 3
NB = 512


def _conv_bn_kernel(ps_ref, w_ref, g_ref, b_ref, rw_ref, psc2_ref):
    w = w_ref[...]
    rw2 = rw_ref[:, C:]
    ts = []
    s = jnp.zeros((C, 1), jnp.float32)
    ss = jnp.zeros((C, 1), jnp.float32)
    for b in range(B):
        t = jnp.dot(w, ps_ref[b], preferred_element_type=jnp.float32)
        ts.append(t)
        s = s + jnp.sum(t, axis=1, keepdims=True)
        ss = ss + jnp.sum(t * t, axis=1, keepdims=True)
    cnt = float(B * M)
    mean = s / cnt
    var = ss / cnt - mean * mean
    scale = g_ref[...] * jax.lax.rsqrt(var + 1e-5)
    for b in range(B):
        t = (ts[b] - mean) * scale + b_ref[...]
        psc = jnp.where(t >= 0, t, 0.2 * t)
        psc2_ref[b] = jnp.dot(rw2, psc, preferred_element_type=jnp.float32)


def _knn_kernel(pcd_ref, ps_ref, psc2_ref, rw_ref, y_ref, st_ref):
    q = pcd_ref[0]
    ps = ps_ref[0]
    qk = jax.lax.dot_general(q, ps, (((0,), (0,)), ((), ())),
                             preferred_element_type=jnp.float32)
    q2 = jnp.sum(q * q, axis=0)[:, None]
    k2 = jnp.sum(ps * ps, axis=0)[None, :]
    dist = jnp.maximum(q2 + k2 - 2.0 * qk, 0.0)

    inf = jnp.float32(jnp.inf)
    a = jnp.full((NB, 128), inf, jnp.float32)
    b = jnp.full((NB, 128), inf, jnp.float32)
    c = jnp.full((NB, 128), inf, jnp.float32)
    for i in range(M // 128):
        x = dist[:, i * 128:(i + 1) * 128]
        t = jnp.maximum(a, x)
        a = jnp.minimum(a, x)
        u = jnp.maximum(b, t)
        b = jnp.minimum(b, t)
        c = jnp.minimum(c, u)
    v1 = jnp.min(a, axis=1, keepdims=True)
    e1 = a == v1
    a2 = jnp.where(e1, b, a)
    b2 = jnp.where(e1, c, b)
    v2 = jnp.min(a2, axis=1, keepdims=True)
    a3 = jnp.where(a2 == v2, b2, a2)
    v3 = jnp.min(a3, axis=1, keepdims=True)

    wsum = (1.0 / (v1 + 1e-8) + 1.0 / (v2 + 1e-8) + 1.0 / (v3 + 1e-8))
    S = jnp.where(dist <= v3, 1.0 / (dist + 1e-8), 0.0)

    y2t = jax.lax.dot_general(S, psc2_ref[0], (((1,), (1,)), ((), ())),
                              preferred_element_type=jnp.float32)
    rw = rw_ref[...]
    y = (jnp.dot(rw[:, :C], q, preferred_element_type=jnp.float32)
         + (y2t / wsum).T)
    y_ref[0] = y

    st = jnp.stack([jnp.sum(y, axis=1), jnp.sum(y * y, axis=1)], axis=1)
    first = pl.program_id(1) == 0

    @pl.when(first)
    def _():
        st_ref[0] = st

    @pl.when(jnp.logical_not(first))
    def _():
        st_ref[0] = st_ref[0] + st


def _bn2_kernel(y_ref, st_ref, g_ref, b_ref, o_ref):
    cnt = float(B * N)
    s = jnp.sum(st_ref[:, :, 0:1], axis=0)
    ss = jnp.sum(st_ref[:, :, 1:2], axis=0)
    mean = s / cnt
    var = ss / cnt - mean * mean
    scale = g_ref[...] * jax.lax.rsqrt(var + 1e-5)
    t = (y_ref[0] - mean) * scale + b_ref[...]
    o_ref[0] = jnp.where(t >= 0, t, 0.2 * t)


def kernel(pcd_up, points_select, idx_select, points_select_xyz, points_drop,
           idx_drop, pcd_up_xyz, conv_w, bn1_g, bn1_b, res_w, bn2_g, bn2_b):
    g1 = bn1_g.reshape(C, 1)
    b1 = bn1_b.reshape(C, 1)
    g2 = bn2_g.reshape(C, 1)
    b2 = bn2_b.reshape(C, 1)

    psc2 = pl.pallas_call(
        _conv_bn_kernel,
        out_shape=jax.ShapeDtypeStruct((B, C, M), jnp.float32),
    )(points_select, conv_w, g1, b1, res_w)

    J = N // NB
    y, st = pl.pallas_call(
        _knn_kernel,
        grid=(B, J),
        compiler_params=pltpu.CompilerParams(
            dimension_semantics=("parallel", "arbitrary")),
        in_specs=[
            pl.BlockSpec((1, C, NB), lambda b, j: (b, 0, j)),
            pl.BlockSpec((1, C, M), lambda b, j: (b, 0, 0)),
            pl.BlockSpec((1, C, M), lambda b, j: (b, 0, 0)),
            pl.BlockSpec((C, 2 * C), lambda b, j: (0, 0)),
        ],
        out_specs=[
            pl.BlockSpec((1, C, NB), lambda b, j: (b, 0, j)),
            pl.BlockSpec((1, C, 2), lambda b, j: (b, 0, 0)),
        ],
        out_shape=[
            jax.ShapeDtypeStruct((B, C, N), jnp.float32),
            jax.ShapeDtypeStruct((B, C, 2), jnp.float32),
        ],
    )(pcd_up, points_select, psc2, res_w)

    x = pl.pallas_call(
        _bn2_kernel,
        grid=(B,),
        in_specs=[
            pl.BlockSpec((1, C, N), lambda b: (b, 0, 0)),
            pl.BlockSpec((B, C, 2), lambda b: (0, 0, 0)),
            pl.BlockSpec((C, 1), lambda b: (0, 0)),
            pl.BlockSpec((C, 1), lambda b: (0, 0)),
        ],
        out_specs=pl.BlockSpec((1, C, N), lambda b: (b, 0, 0)),
        out_shape=jax.ShapeDtypeStruct((B, C, N), jnp.float32),
    )(y, st, g2, b2)
    return x


# speedup vs baseline: 1.3479x; 1.0916x over previous
import jax
import jax.numpy as jnp
from jax.experimental import pallas as pl
from jax.experimental.pallas import tpu as pltpu

B, C, N, M, K = 4, 64, 8192, 2048, 3
NB = 1024


def _conv_bn_kernel(ps_ref, w_ref, g_ref, b_ref, rw_ref, psc2_ref):
    w = w_ref[...]
    rw2 = rw_ref[:, C:]
    ts = []
    s = jnp.zeros((C, 1), jnp.float32)
    ss = jnp.zeros((C, 1), jnp.float32)
    for b in range(B):
        t = jnp.dot(w, ps_ref[b], preferred_element_type=jnp.float32)
        ts.append(t)
        s = s + jnp.sum(t, axis=1, keepdims=True)
        ss = ss + jnp.sum(t * t, axis=1, keepdims=True)
    cnt = float(B * M)
    mean = s / cnt
    var = ss / cnt - mean * mean
    scale = g_ref[...] * jax.lax.rsqrt(var + 1e-5)
    for b in range(B):
        t = (ts[b] - mean) * scale + b_ref[...]
        psc = jnp.where(t >= 0, t, 0.2 * t)
        psc2_ref[b] = jnp.dot(rw2, psc, preferred_element_type=jnp.float32)


def _knn_kernel(pcd_ref, ps_ref, psc2_ref, rw_ref, y_ref, st_ref):
    q = pcd_ref[0]
    ps = ps_ref[0]
    qk = jax.lax.dot_general(q, ps, (((0,), (0,)), ((), ())),
                             preferred_element_type=jnp.float32)
    q2 = jnp.sum(q * q, axis=0)[:, None]
    k2 = jnp.sum(ps * ps, axis=0)[None, :]
    dist = jnp.maximum(q2 + k2 - 2.0 * qk, 0.0)

    inf = jnp.float32(jnp.inf)
    a = jnp.full((NB, 128), inf, jnp.float32)
    b = jnp.full((NB, 128), inf, jnp.float32)
    c = jnp.full((NB, 128), inf, jnp.float32)
    for i in range(M // 128):
        x = dist[:, i * 128:(i + 1) * 128]
        t = jnp.maximum(a, x)
        a = jnp.minimum(a, x)
        u = jnp.maximum(b, t)
        b = jnp.minimum(b, t)
        c = jnp.minimum(c, u)
    v1 = jnp.min(a, axis=1, keepdims=True)
    e1 = a == v1
    a2 = jnp.where(e1, b, a)
    b2 = jnp.where(e1, c, b)
    v2 = jnp.min(a2, axis=1, keepdims=True)
    a3 = jnp.where(a2 == v2, b2, a2)
    v3 = jnp.min(a3, axis=1, keepdims=True)

    wsum = (1.0 / (v1 + 1e-8) + 1.0 / (v2 + 1e-8) + 1.0 / (v3 + 1e-8))
    S = jnp.where(dist <= v3, 1.0 / (dist + 1e-8), 0.0)

    y2t = jax.lax.dot_general(S, psc2_ref[0], (((1,), (1,)), ((), ())),
                              preferred_element_type=jnp.float32)
    rw = rw_ref[...]
    y = (jnp.dot(rw[:, :C], q, preferred_element_type=jnp.float32)
         + (y2t / wsum).T)
    y_ref[0] = y

    st = jnp.stack([jnp.sum(y, axis=1), jnp.sum(y * y, axis=1)], axis=1)
    first = pl.program_id(1) == 0

    @pl.when(first)
    def _():
        st_ref[0] = st

    @pl.when(jnp.logical_not(first))
    def _():
        st_ref[0] = st_ref[0] + st


def _bn2_kernel(y_ref, st_ref, g_ref, b_ref, o_ref):
    cnt = float(B * N)
    s = jnp.sum(st_ref[:, :, 0:1], axis=0)
    ss = jnp.sum(st_ref[:, :, 1:2], axis=0)
    mean = s / cnt
    var = ss / cnt - mean * mean
    scale = g_ref[...] * jax.lax.rsqrt(var + 1e-5)
    t = (y_ref[0] - mean) * scale + b_ref[...]
    o_ref[0] = jnp.where(t >= 0, t, 0.2 * t)


def kernel(pcd_up, points_select, idx_select, points_select_xyz, points_drop,
           idx_drop, pcd_up_xyz, conv_w, bn1_g, bn1_b, res_w, bn2_g, bn2_b):
    g1 = bn1_g.reshape(C, 1)
    b1 = bn1_b.reshape(C, 1)
    g2 = bn2_g.reshape(C, 1)
    b2 = bn2_b.reshape(C, 1)

    psc2 = pl.pallas_call(
        _conv_bn_kernel,
        out_shape=jax.ShapeDtypeStruct((B, C, M), jnp.float32),
    )(points_select, conv_w, g1, b1, res_w)

    J = N // NB
    y, st = pl.pallas_call(
        _knn_kernel,
        grid=(B, J),
        compiler_params=pltpu.CompilerParams(
            dimension_semantics=("parallel", "arbitrary")),
        in_specs=[
            pl.BlockSpec((1, C, NB), lambda b, j: (b, 0, j)),
            pl.BlockSpec((1, C, M), lambda b, j: (b, 0, 0)),
            pl.BlockSpec((1, C, M), lambda b, j: (b, 0, 0)),
            pl.BlockSpec((C, 2 * C), lambda b, j: (0, 0)),
        ],
        out_specs=[
            pl.BlockSpec((1, C, NB), lambda b, j: (b, 0, j)),
            pl.BlockSpec((1, C, 2), lambda b, j: (b, 0, 0)),
        ],
        out_shape=[
            jax.ShapeDtypeStruct((B, C, N), jnp.float32),
            jax.ShapeDtypeStruct((B, C, 2), jnp.float32),
        ],
    )(pcd_up, points_select, psc2, res_w)

    x = pl.pallas_call(
        _bn2_kernel,
        grid=(B,),
        in_specs=[
            pl.BlockSpec((1, C, N), lambda b: (b, 0, 0)),
            pl.BlockSpec((B, C, 2), lambda b: (0, 0, 0)),
            pl.BlockSpec((C, 1), lambda b: (0, 0)),
            pl.BlockSpec((C, 1), lambda b: (0, 0)),
        ],
        out_specs=pl.BlockSpec((1, C, N), lambda b: (b, 0, 0)),
        out_shape=jax.ShapeDtypeStruct((B, C, N), jnp.float32),
    )(y, st, g2, b2)
    return x


# speedup vs baseline: 1.4236x; 1.0561x over previous
import jax
import jax.numpy as jnp
from jax.experimental import pallas as pl
from jax.experimental.pallas import tpu as pltpu

B, C, N, M, K = 4, 64, 8192, 2048, 3
NB = 2048


def _conv_bn_kernel(ps_ref, w_ref, g_ref, b_ref, rw_ref, psc2_ref):
    w = w_ref[...]
    rw2 = rw_ref[:, C:]
    ts = []
    s = jnp.zeros((C, 1), jnp.float32)
    ss = jnp.zeros((C, 1), jnp.float32)
    for b in range(B):
        t = jnp.dot(w, ps_ref[b], preferred_element_type=jnp.float32)
        ts.append(t)
        s = s + jnp.sum(t, axis=1, keepdims=True)
        ss = ss + jnp.sum(t * t, axis=1, keepdims=True)
    cnt = float(B * M)
    mean = s / cnt
    var = ss / cnt - mean * mean
    scale = g_ref[...] * jax.lax.rsqrt(var + 1e-5)
    for b in range(B):
        t = (ts[b] - mean) * scale + b_ref[...]
        psc = jnp.where(t >= 0, t, 0.2 * t)
        psc2_ref[b] = jnp.dot(rw2, psc, preferred_element_type=jnp.float32)


def _knn_kernel(pcd_ref, ps_ref, psc2_ref, rw_ref, y_ref, st_ref):
    q = pcd_ref[0]
    ps = ps_ref[0]
    qk = jax.lax.dot_general(q, ps, (((0,), (0,)), ((), ())),
                             preferred_element_type=jnp.float32)
    q2 = jnp.sum(q * q, axis=0)[:, None]
    k2 = jnp.sum(ps * ps, axis=0)[None, :]
    dist = jnp.maximum(q2 + k2 - 2.0 * qk, 0.0)

    inf = jnp.float32(jnp.inf)
    a = jnp.full((NB, 128), inf, jnp.float32)
    b = jnp.full((NB, 128), inf, jnp.float32)
    c = jnp.full((NB, 128), inf, jnp.float32)
    for i in range(M // 128):
        x = dist[:, i * 128:(i + 1) * 128]
        t = jnp.maximum(a, x)
        a = jnp.minimum(a, x)
        u = jnp.maximum(b, t)
        b = jnp.minimum(b, t)
        c = jnp.minimum(c, u)
    v1 = jnp.min(a, axis=1, keepdims=True)
    e1 = a == v1
    a2 = jnp.where(e1, b, a)
    b2 = jnp.where(e1, c, b)
    v2 = jnp.min(a2, axis=1, keepdims=True)
    a3 = jnp.where(a2 == v2, b2, a2)
    v3 = jnp.min(a3, axis=1, keepdims=True)

    wsum = (1.0 / (v1 + 1e-8) + 1.0 / (v2 + 1e-8) + 1.0 / (v3 + 1e-8))
    S = jnp.where(dist <= v3, 1.0 / (dist + 1e-8), 0.0)

    y2t = jax.lax.dot_general(S, psc2_ref[0], (((1,), (1,)), ((), ())),
                              preferred_element_type=jnp.float32)
    rw = rw_ref[...]
    y = (jnp.dot(rw[:, :C], q, preferred_element_type=jnp.float32)
         + (y2t / wsum).T)
    y_ref[0] = y

    st = jnp.stack([jnp.sum(y, axis=1), jnp.sum(y * y, axis=1)], axis=1)
    first = pl.program_id(1) == 0

    @pl.when(first)
    def _():
        st_ref[0] = st

    @pl.when(jnp.logical_not(first))
    def _():
        st_ref[0] = st_ref[0] + st


def _bn2_kernel(y_ref, st_ref, g_ref, b_ref, o_ref):
    cnt = float(B * N)
    s = jnp.sum(st_ref[:, :, 0:1], axis=0)
    ss = jnp.sum(st_ref[:, :, 1:2], axis=0)
    mean = s / cnt
    var = ss / cnt - mean * mean
    scale = g_ref[...] * jax.lax.rsqrt(var + 1e-5)
    t = (y_ref[0] - mean) * scale + b_ref[...]
    o_ref[0] = jnp.where(t >= 0, t, 0.2 * t)


def kernel(pcd_up, points_select, idx_select, points_select_xyz, points_drop,
           idx_drop, pcd_up_xyz, conv_w, bn1_g, bn1_b, res_w, bn2_g, bn2_b):
    g1 = bn1_g.reshape(C, 1)
    b1 = bn1_b.reshape(C, 1)
    g2 = bn2_g.reshape(C, 1)
    b2 = bn2_b.reshape(C, 1)

    psc2 = pl.pallas_call(
        _conv_bn_kernel,
        out_shape=jax.ShapeDtypeStruct((B, C, M), jnp.float32),
    )(points_select, conv_w, g1, b1, res_w)

    J = N // NB
    y, st = pl.pallas_call(
        _knn_kernel,
        grid=(B, J),
        compiler_params=pltpu.CompilerParams(
            dimension_semantics=("parallel", "arbitrary")),
        in_specs=[
            pl.BlockSpec((1, C, NB), lambda b, j: (b, 0, j)),
            pl.BlockSpec((1, C, M), lambda b, j: (b, 0, 0)),
            pl.BlockSpec((1, C, M), lambda b, j: (b, 0, 0)),
            pl.BlockSpec((C, 2 * C), lambda b, j: (0, 0)),
        ],
        out_specs=[
            pl.BlockSpec((1, C, NB), lambda b, j: (b, 0, j)),
            pl.BlockSpec((1, C, 2), lambda b, j: (b, 0, 0)),
        ],
        out_shape=[
            jax.ShapeDtypeStruct((B, C, N), jnp.float32),
            jax.ShapeDtypeStruct((B, C, 2), jnp.float32),
        ],
    )(pcd_up, points_select, psc2, res_w)

    x = pl.pallas_call(
        _bn2_kernel,
        grid=(B,),
        in_specs=[
            pl.BlockSpec((1, C, N), lambda b: (b, 0, 0)),
            pl.BlockSpec((B, C, 2), lambda b: (0, 0, 0)),
            pl.BlockSpec((C, 1), lambda b: (0, 0)),
            pl.BlockSpec((C, 1), lambda b: (0, 0)),
        ],
        out_specs=pl.BlockSpec((1, C, N), lambda b: (b, 0, 0)),
        out_shape=jax.ShapeDtypeStruct((B, C, N), jnp.float32),
    )(y, st, g2, b2)
    return x
